# baseline (speedup 1.0000x reference)
import functools

import jax
import jax.numpy as jnp
from jax import lax
from jax.experimental import pallas as pl
from jax.experimental.pallas import tpu as pltpu
from jax.experimental.pallas import tpu_sc as plsc

_K = 1024
_D = 256
_R = 2048
_BIG = 1.0e8


def _vq_body(z_ref, w_ref, idx_ref, loss_ref):
    g = pl.program_id(0)
    z = z_ref[...]
    w = w_ref[...]

    sumz = jnp.sum(z * z, axis=1, keepdims=True)
    ones_row = jnp.ones((1, _D), dtype=jnp.float32)
    sumw = jax.lax.dot_general(ones_row, w * w, (((1,), (1,)), ((), ())),
                               preferred_element_type=jnp.float32)
    mm = jax.lax.dot_general(z, w, (((1,), (1,)), ((), ())),
                             preferred_element_type=jnp.float32)
    dmat = (sumz + sumw) - 2.0 * mm

    minval = jnp.min(dmat, axis=1, keepdims=True)
    kiota_i = jax.lax.broadcasted_iota(jnp.int32, dmat.shape, 1)
    key = (dmat - minval) * _BIG + kiota_i.astype(jnp.float32)
    minkey = jnp.min(key, axis=1, keepdims=True)
    idx = minkey.astype(jnp.int32)
    idx_ref[0] = jnp.transpose(idx)

    part = jnp.sum(minval, keepdims=True)

    @pl.when(g == 0)
    def _():
        loss_ref[...] = jnp.zeros_like(part)

    loss_ref[...] += part


def _make_sc_gather(M, D):
    info = plsc.get_sparse_core_info()
    NW = info.num_cores * info.num_subcores
    b_per_w = M // NW
    chunk = 128
    n_chunks = b_per_w // chunk
    mesh = plsc.VectorSubcoreMesh(core_axis_name="c", subcore_axis_name="s")

    @functools.partial(
        pl.kernel, mesh=mesh,
        out_type=jax.ShapeDtypeStruct((M, D), jnp.float32),
        scratch_types=[
            pltpu.VMEM((chunk,), jnp.int32),
            pltpu.VMEM((chunk, D), jnp.float32),
            pltpu.SemaphoreType.DMA,
        ],
    )
    def gather(table_hbm, idx_hbm, out_hbm, idx_v, rows_v, sem):
        wid = lax.axis_index("s") * info.num_cores + lax.axis_index("c")
        base = wid * b_per_w

        def body(c, _):
            off = base + c * chunk
            pltpu.sync_copy(idx_hbm.at[pl.ds(off, chunk)], idx_v)
            pltpu.async_copy(table_hbm.at[idx_v], rows_v, sem).wait()
            pltpu.sync_copy(rows_v, out_hbm.at[pl.ds(off, chunk)])
            return _

        lax.fori_loop(0, n_chunks, body, 0)

    return gather


def kernel(z_e, W):
    B, D, H, Wd = z_e.shape
    N = H * Wd
    M = B * N
    z2 = jnp.transpose(z_e, (0, 2, 3, 1)).reshape(M, D)

    idx3, loss_sum = pl.pallas_call(
        _vq_body,
        grid=(M // _R,),
        in_specs=[
            pl.BlockSpec((_R, D), lambda g: (g, 0)),
            pl.BlockSpec((_K, D), lambda g: (0, 0)),
        ],
        out_specs=[
            pl.BlockSpec((1, 1, _R), lambda g: (g, 0, 0)),
            pl.BlockSpec((1, 1), lambda g: (0, 0)),
        ],
        out_shape=[
            jax.ShapeDtypeStruct((M // _R, 1, _R), jnp.int32),
            jax.ShapeDtypeStruct((1, 1), jnp.float32),
        ],
    )(z2, W)

    idx_flat = idx3.reshape(M)
    zq2 = _make_sc_gather(M, D)(W, idx_flat)

    z_q_st = jnp.transpose(zq2.reshape(B, H, Wd, D), (0, 3, 1, 2))
    indices = idx3.reshape(B, H, Wd)
    loss = loss_sum[0, 0] / (M * D)
    return (z_q_st, loss, loss, indices)


# speedup vs baseline: 1.0103x; 1.0103x over previous
import functools

import jax
import jax.numpy as jnp
from jax import lax
from jax.experimental import pallas as pl
from jax.experimental.pallas import tpu as pltpu
from jax.experimental.pallas import tpu_sc as plsc

_K = 1024
_D = 256
_R = 2048
_BIG = 1.0e8


def _vq_body(z_ref, w_ref, idx_ref, loss_ref):
    g = pl.program_id(0)
    z = z_ref[...]
    w = w_ref[...]

    sumz = jnp.sum(z * z, axis=1, keepdims=True)
    ones_row = jnp.ones((1, _D), dtype=jnp.float32)
    sumw = jax.lax.dot_general(ones_row, w * w, (((1,), (1,)), ((), ())),
                               preferred_element_type=jnp.float32)
    mm = jax.lax.dot_general(z, w, (((1,), (1,)), ((), ())),
                             preferred_element_type=jnp.float32)
    dmat = (sumz + sumw) - 2.0 * mm

    minval = jnp.min(dmat, axis=1, keepdims=True)
    kiota_i = jax.lax.broadcasted_iota(jnp.int32, dmat.shape, 1)
    key = (dmat - minval) * _BIG + kiota_i.astype(jnp.float32)
    minkey = jnp.min(key, axis=1, keepdims=True)
    idx = minkey.astype(jnp.int32)
    idx_ref[0] = jnp.transpose(idx)

    part = jnp.sum(minval, keepdims=True)

    @pl.when(g == 0)
    def _():
        loss_ref[...] = jnp.zeros_like(part)

    loss_ref[...] += part


def _make_sc_gather(M, D):
    info = plsc.get_sparse_core_info()
    NW = info.num_cores * info.num_subcores
    b_per_w = M // NW
    chunk = 128
    n_chunks = b_per_w // chunk
    mesh = plsc.VectorSubcoreMesh(core_axis_name="c", subcore_axis_name="s")

    @functools.partial(
        pl.kernel, mesh=mesh,
        out_type=jax.ShapeDtypeStruct((M, D), jnp.float32),
        scratch_types=[
            pltpu.VMEM((b_per_w,), jnp.int32),
            pltpu.VMEM((chunk, D), jnp.float32),
            pltpu.VMEM((chunk, D), jnp.float32),
            pltpu.SemaphoreType.DMA,
            pltpu.SemaphoreType.DMA,
            pltpu.SemaphoreType.DMA,
            pltpu.SemaphoreType.DMA,
        ],
    )
    def gather(table_hbm, idx_hbm, out_hbm, idx_v, buf0, buf1,
               sg0, sg1, ss0, ss1):
        wid = lax.axis_index("s") * info.num_cores + lax.axis_index("c")
        base = wid * b_per_w
        pltpu.sync_copy(idx_hbm.at[pl.ds(base, b_per_w)], idx_v)

        bufs = (buf0, buf1)
        sgs = (sg0, sg1)
        sss = (ss0, ss1)

        def start_g(c):
            return pltpu.async_copy(
                table_hbm.at[idx_v.at[pl.ds(c * chunk, chunk)]],
                bufs[c % 2], sgs[c % 2])

        def start_s(c):
            return pltpu.async_copy(
                bufs[c % 2], out_hbm.at[pl.ds(base + c * chunk, chunk)],
                sss[c % 2])

        g = [start_g(0), start_g(1)]
        s = [None, None]
        for c in range(n_chunks):
            g[c % 2].wait()
            s[c % 2] = start_s(c)
            if c + 2 < n_chunks:
                s[c % 2].wait()
                g[c % 2] = start_g(c + 2)
        s[0].wait()
        s[1].wait()

    return gather


def kernel(z_e, W):
    B, D, H, Wd = z_e.shape
    N = H * Wd
    M = B * N
    z2 = jnp.transpose(z_e, (0, 2, 3, 1)).reshape(M, D)

    idx3, loss_sum = pl.pallas_call(
        _vq_body,
        grid=(M // _R,),
        in_specs=[
            pl.BlockSpec((_R, D), lambda g: (g, 0)),
            pl.BlockSpec((_K, D), lambda g: (0, 0)),
        ],
        out_specs=[
            pl.BlockSpec((1, 1, _R), lambda g: (g, 0, 0)),
            pl.BlockSpec((1, 1), lambda g: (0, 0)),
        ],
        out_shape=[
            jax.ShapeDtypeStruct((M // _R, 1, _R), jnp.int32),
            jax.ShapeDtypeStruct((1, 1), jnp.float32),
        ],
    )(z2, W)

    idx_flat = idx3.reshape(M)
    zq2 = _make_sc_gather(M, D)(W, idx_flat)

    z_q_st = jnp.transpose(zq2.reshape(B, H, Wd, D), (0, 3, 1, 2))
    indices = idx3.reshape(B, H, Wd)
    loss = loss_sum[0, 0] / (M * D)
    return (z_q_st, loss, loss, indices)


# speedup vs baseline: 1.2307x; 1.2182x over previous
import jax
import jax.numpy as jnp
from jax.experimental import pallas as pl

_K = 1024
_D = 256
_R = 4096
_BIG = 1.0e8


def _vq_body(z_ref, w_ref, zq_ref, idx_ref, loss_ref):
    g = pl.program_id(0)
    z = z_ref[...]
    w = w_ref[...]

    sumz = jnp.sum(z * z, axis=1, keepdims=True)
    ones_row = jnp.ones((1, _D), dtype=jnp.float32)
    sumw = jax.lax.dot_general(ones_row, w * w, (((1,), (1,)), ((), ())),
                               preferred_element_type=jnp.float32)
    mm = jax.lax.dot_general(z, w, (((1,), (1,)), ((), ())),
                             preferred_element_type=jnp.float32)
    dmat = (sumz + sumw) - 2.0 * mm

    minval = jnp.min(dmat, axis=1, keepdims=True)
    kiota_i = jax.lax.broadcasted_iota(jnp.int32, dmat.shape, 1)
    key = (dmat - minval) * _BIG + kiota_i.astype(jnp.float32)
    minkey = jnp.min(key, axis=1, keepdims=True)
    idx = minkey.astype(jnp.int32)
    idx_ref[0] = jnp.transpose(idx)

    onehot = (kiota_i == idx).astype(jnp.float32)
    zq = jax.lax.dot_general(onehot, w, (((1,), (0,)), ((), ())),
                             preferred_element_type=jnp.float32)
    zq_ref[...] = zq

    part = jnp.sum(minval, keepdims=True)

    @pl.when(g == 0)
    def _():
        loss_ref[...] = jnp.zeros_like(part)

    loss_ref[...] += part


def kernel(z_e, W):
    B, D, H, Wd = z_e.shape
    N = H * Wd
    M = B * N
    z2 = jnp.transpose(z_e, (0, 2, 3, 1)).reshape(M, D)

    zq2, idx3, loss_sum = pl.pallas_call(
        _vq_body,
        grid=(M // _R,),
        in_specs=[
            pl.BlockSpec((_R, D), lambda g: (g, 0)),
            pl.BlockSpec((_K, D), lambda g: (0, 0)),
        ],
        out_specs=[
            pl.BlockSpec((_R, D), lambda g: (g, 0)),
            pl.BlockSpec((1, 1, _R), lambda g: (g, 0, 0)),
            pl.BlockSpec((1, 1), lambda g: (0, 0)),
        ],
        out_shape=[
            jax.ShapeDtypeStruct((M, D), jnp.float32),
            jax.ShapeDtypeStruct((M // _R, 1, _R), jnp.int32),
            jax.ShapeDtypeStruct((1, 1), jnp.float32),
        ],
    )(z2, W)

    z_q_st = jnp.transpose(zq2.reshape(B, H, Wd, D), (0, 3, 1, 2))
    indices = idx3.reshape(B, H, Wd)
    loss = loss_sum[0, 0] / (M * D)
    return (z_q_st, loss, loss, indices)
